# baseline (device time: 59492 ns/iter reference)
import functools

import jax
import jax.numpy as jnp
from jax import lax
from jax.experimental import pallas as pl
from jax.experimental.pallas import tpu as pltpu

N_DEV = 8


def kernel(x, w_mat):
    m_per, k = x.shape
    _, n_per = w_mat.shape

    def body(x_ref, w_ref, out_ref, xg_ref, send_sems, recv_sems):
        my = lax.axis_index("i")
        left = lax.rem(my - 1 + N_DEV, N_DEV)
        right = lax.rem(my + 1, N_DEV)

        barrier_sem = pltpu.get_barrier_semaphore()
        for nbr in (left, right):
            pl.semaphore_signal(
                barrier_sem, inc=1,
                device_id=(nbr,), device_id_type=pl.DeviceIdType.MESH,
            )
        pl.semaphore_wait(barrier_sem, 2)

        xg_ref[pl.ds(my * m_per, m_per), :] = x_ref[...]

        for h in range(N_DEV - 1):
            origin = lax.rem(my - h + N_DEV, N_DEV)
            rdma = pltpu.make_async_remote_copy(
                src_ref=xg_ref.at[pl.ds(origin * m_per, m_per), :],
                dst_ref=xg_ref.at[pl.ds(origin * m_per, m_per), :],
                send_sem=send_sems.at[h],
                recv_sem=recv_sems.at[h],
                device_id=(right,),
                device_id_type=pl.DeviceIdType.MESH,
            )
            rdma.start()
            rdma.wait()

        out_ref[...] = jnp.maximum(
            jnp.dot(xg_ref[...], w_ref[...],
                    preferred_element_type=jnp.float32),
            0.0,
        )

        @functools.partial(pl.run_scoped, exit_sem=pltpu.SemaphoreType.REGULAR)
        def _(exit_sem):
            pl.semaphore_signal(
                exit_sem, inc=1,
                device_id=(left,), device_id_type=pl.DeviceIdType.MESH,
            )
            pl.semaphore_wait(exit_sem, 1)

    return pl.pallas_call(
        body,
        out_shape=jax.ShapeDtypeStruct((N_DEV * m_per, n_per), jnp.float32),
        in_specs=[
            pl.BlockSpec(memory_space=pltpu.VMEM),
            pl.BlockSpec(memory_space=pltpu.VMEM),
        ],
        out_specs=pl.BlockSpec(memory_space=pltpu.VMEM),
        scratch_shapes=[
            pltpu.VMEM((N_DEV * m_per, k), jnp.float32),
            pltpu.SemaphoreType.DMA((N_DEV - 1,)),
            pltpu.SemaphoreType.DMA((N_DEV - 1,)),
        ],
        compiler_params=pltpu.CompilerParams(collective_id=0),
    )(x, w_mat)


# device time: 29276 ns/iter; 2.0321x vs baseline; 2.0321x over previous
import jax
import jax.numpy as jnp
from jax import lax
from jax.experimental import pallas as pl
from jax.experimental.pallas import tpu as pltpu

N_DEV = 8
N_MSG = 7


def kernel(x, w_mat):
    m_per, k = x.shape
    _, n_per = w_mat.shape
    p = m_per // 2

    def body(x_ref, w_ref, out_ref, xg_ref, fs_sems, fr_sems, bs_sems, br_sems):
        my = lax.axis_index("i")
        left = lax.rem(my - 1 + N_DEV, N_DEV)
        right = lax.rem(my + 1, N_DEV)

        def o(kk):
            return lax.rem(my - kk + N_DEV, N_DEV)

        def q(kk):
            return lax.rem(my + kk, N_DEV)

        def piece(pidx):
            return xg_ref.at[pl.ds(pidx * p, p), :]

        def send(src, pidx, dev, ssem, rsem):
            d = pltpu.make_async_remote_copy(
                src_ref=src, dst_ref=piece(pidx),
                send_sem=ssem, recv_sem=rsem,
                device_id=(dev,), device_id_type=pl.DeviceIdType.MESH,
            )
            d.start()
            return d

        def recv_wait(pidx, ssem, rsem):
            pltpu.make_async_remote_copy(
                src_ref=piece(pidx), dst_ref=piece(pidx),
                send_sem=ssem, recv_sem=rsem,
                device_id=(right,), device_id_type=pl.DeviceIdType.MESH,
            ).wait_recv()

        def gemm(row0, nrows, src):
            out_ref[pl.ds(row0, nrows), :] = jnp.maximum(
                jnp.dot(src, w_ref[...], preferred_element_type=jnp.float32),
                0.0,
            )

        barrier_sem = pltpu.get_barrier_semaphore()
        for nbr in (left, right):
            pl.semaphore_signal(
                barrier_sem, inc=1,
                device_id=(nbr,), device_id_type=pl.DeviceIdType.MESH,
            )
        pl.semaphore_wait(barrier_sem, 2)

        fsend = [2 * o(0), 2 * o(0) + 1, 2 * o(1), 2 * o(1) + 1,
                 2 * o(2), 2 * o(2) + 1, 2 * o(3)]
        frecv = [2 * o(1), 2 * o(1) + 1, 2 * o(2), 2 * o(2) + 1,
                 2 * o(3), 2 * o(3) + 1, 2 * o(4)]
        bsend = [2 * q(0) + 1, 2 * q(0), 2 * q(1) + 1, 2 * q(1),
                 2 * q(2) + 1, 2 * q(2), 2 * q(3) + 1]
        brecv = [2 * q(1) + 1, 2 * q(1), 2 * q(2) + 1, 2 * q(2),
                 2 * q(3) + 1, 2 * q(3), 2 * q(4) + 1]

        started = []
        started.append(send(x_ref.at[pl.ds(0, p), :], fsend[0], right,
                            fs_sems.at[0], fr_sems.at[0]))
        started.append(send(x_ref.at[pl.ds(p, p), :], fsend[1], right,
                            fs_sems.at[1], fr_sems.at[1]))
        started.append(send(x_ref.at[pl.ds(p, p), :], bsend[0], left,
                            bs_sems.at[0], br_sems.at[0]))
        started.append(send(x_ref.at[pl.ds(0, p), :], bsend[1], left,
                            bs_sems.at[1], br_sems.at[1]))

        gemm(my * m_per, m_per, x_ref[...])

        for j in range(N_MSG):
            recv_wait(frecv[j], fs_sems.at[j], fr_sems.at[j])
            if j + 2 < N_MSG:
                started.append(send(piece(fsend[j + 2]), fsend[j + 2], right,
                                    fs_sems.at[j + 2], fr_sems.at[j + 2]))
            recv_wait(brecv[j], bs_sems.at[j], br_sems.at[j])
            if j + 2 < N_MSG:
                started.append(send(piece(bsend[j + 2]), bsend[j + 2], left,
                                    bs_sems.at[j + 2], br_sems.at[j + 2]))
            gemm(frecv[j] * p, p, piece(frecv[j])[...])
            gemm(brecv[j] * p, p, piece(brecv[j])[...])

        for d in started:
            d.wait_send()

    return pl.pallas_call(
        body,
        out_shape=jax.ShapeDtypeStruct((N_DEV * m_per, n_per), jnp.float32),
        in_specs=[
            pl.BlockSpec(memory_space=pltpu.VMEM),
            pl.BlockSpec(memory_space=pltpu.VMEM),
        ],
        out_specs=pl.BlockSpec(memory_space=pltpu.VMEM),
        scratch_shapes=[
            pltpu.VMEM((N_DEV * m_per, k), jnp.float32),
            pltpu.SemaphoreType.DMA((N_MSG,)),
            pltpu.SemaphoreType.DMA((N_MSG,)),
            pltpu.SemaphoreType.DMA((N_MSG,)),
            pltpu.SemaphoreType.DMA((N_MSG,)),
        ],
        compiler_params=pltpu.CompilerParams(collective_id=0),
    )(x, w_mat)


# device time: 23553 ns/iter; 2.5259x vs baseline; 1.2430x over previous
import jax
import jax.numpy as jnp
from jax import lax
from jax.experimental import pallas as pl
from jax.experimental.pallas import tpu as pltpu

N_DEV = 8
NR = 5
NE = 4


def kernel(x, w_mat):
    m_per, k = x.shape
    _, n_per = w_mat.shape
    p = m_per // 2

    def body(x_ref, w_ref, out_ref, xg_ref, fs, fr, bs, br, es, er):
        l = lax.axis_index("i")
        pos = jnp.where(l < 4, l, 11 - l)
        parity = lax.rem(pos, 2)
        s = 1 - 2 * parity

        def l_of(qq):
            qq = lax.rem(qq + 2 * N_DEV, N_DEV)
            return jnp.where(qq < 4, qq, 11 - qq)

        right = l_of(pos + 1)
        left = l_of(pos - 1)
        prt = l_of(pos + 3 * s)

        def o(kk):
            return l_of(pos - kk)

        def q_(kk):
            return l_of(pos + kk)

        def piece(pidx):
            return xg_ref.at[pl.ds(pidx * p, p), :]

        def send(src, pidx, dev, ssem, rsem):
            d = pltpu.make_async_remote_copy(
                src_ref=src, dst_ref=piece(pidx),
                send_sem=ssem, recv_sem=rsem,
                device_id=(dev,), device_id_type=pl.DeviceIdType.MESH,
            )
            d.start()
            return d

        def recv_wait(pidx, ssem, rsem):
            pltpu.make_async_remote_copy(
                src_ref=piece(pidx), dst_ref=piece(pidx),
                send_sem=ssem, recv_sem=rsem,
                device_id=(right,), device_id_type=pl.DeviceIdType.MESH,
            ).wait_recv()

        def gemm_piece(pidx):
            out_ref[pl.ds(pidx * p, p), :] = jnp.maximum(
                jnp.dot(piece(pidx)[...], w_ref[...],
                        preferred_element_type=jnp.float32),
                0.0,
            )

        barrier_sem = pltpu.get_barrier_semaphore()
        for nbr in (left, right, prt):
            pl.semaphore_signal(
                barrier_sem, inc=1,
                device_id=(nbr,), device_id_type=pl.DeviceIdType.MESH,
            )
        pl.semaphore_wait(barrier_sem, 3)

        fsendp = [2 * o(0), 2 * o(0) + 1, 2 * o(1), 2 * o(1) + 1, 2 * o(2)]
        frecvp = [2 * o(1), 2 * o(1) + 1, 2 * o(2), 2 * o(2) + 1, 2 * o(3)]
        bsendp = [2 * q_(0) + 1, 2 * q_(0), 2 * q_(1) + 1, 2 * q_(1),
                  2 * q_(2) + 1]
        brecvp = [2 * q_(1) + 1, 2 * q_(1), 2 * q_(2) + 1, 2 * q_(2),
                  2 * q_(3) + 1]
        c1 = l_of(pos - s)
        c2 = l_of(pos - 2 * s)
        esendp = [2 * l + 1 - parity,
                  2 * c1 + parity,
                  2 * c1 + 1 - parity,
                  2 * c2 + parity]
        a = l_of(pos + 4)
        erecvp = [2 * prt + parity,
                  2 * a + 1 - parity,
                  2 * a + parity,
                  2 * l_of(pos - 3 * s) + 1 - parity]

        started = []
        xt = x_ref.at[pl.ds(0, p), :]
        xb = x_ref.at[pl.ds(p, p), :]
        started.append(send(xt, fsendp[0], right, fs.at[0], fr.at[0]))
        started.append(send(xb, fsendp[1], right, fs.at[1], fr.at[1]))
        started.append(send(xb, bsendp[0], left, bs.at[0], br.at[0]))
        started.append(send(xt, bsendp[1], left, bs.at[1], br.at[1]))
        started.append(send(x_ref.at[pl.ds((1 - parity) * p, p), :],
                            esendp[0], prt, es.at[0], er.at[0]))

        out_ref[pl.ds(l * m_per, m_per), :] = jnp.maximum(
            jnp.dot(x_ref[...], w_ref[...],
                    preferred_element_type=jnp.float32),
            0.0,
        )

        for j in range(NR):
            recv_wait(frecvp[j], fs.at[j], fr.at[j])
            if j + 2 < NR:
                started.append(send(piece(fsendp[j + 2]), fsendp[j + 2],
                                    right, fs.at[j + 2], fr.at[j + 2]))
            recv_wait(brecvp[j], bs.at[j], br.at[j])
            if j + 2 < NR:
                started.append(send(piece(bsendp[j + 2]), bsendp[j + 2],
                                    left, bs.at[j + 2], br.at[j + 2]))
            if j < 3:
                started.append(send(piece(esendp[j + 1]), esendp[j + 1],
                                    prt, es.at[j + 1], er.at[j + 1]))
            gemm_piece(frecvp[j])
            gemm_piece(brecvp[j])

        for t in range(NE):
            recv_wait(erecvp[t], es.at[t], er.at[t])
            gemm_piece(erecvp[t])

        for d in started:
            d.wait_send()

    return pl.pallas_call(
        body,
        out_shape=jax.ShapeDtypeStruct((N_DEV * m_per, n_per), jnp.float32),
        in_specs=[
            pl.BlockSpec(memory_space=pltpu.VMEM),
            pl.BlockSpec(memory_space=pltpu.VMEM),
        ],
        out_specs=pl.BlockSpec(memory_space=pltpu.VMEM),
        scratch_shapes=[
            pltpu.VMEM((N_DEV * m_per, k), jnp.float32),
            pltpu.SemaphoreType.DMA((NR,)),
            pltpu.SemaphoreType.DMA((NR,)),
            pltpu.SemaphoreType.DMA((NR,)),
            pltpu.SemaphoreType.DMA((NR,)),
            pltpu.SemaphoreType.DMA((NE,)),
            pltpu.SemaphoreType.DMA((NE,)),
        ],
        compiler_params=pltpu.CompilerParams(collective_id=0),
    )(x, w_mat)


# device time: 22411 ns/iter; 2.6546x vs baseline; 1.0510x over previous
import jax
import jax.numpy as jnp
from jax import lax
from jax.experimental import pallas as pl
from jax.experimental.pallas import tpu as pltpu

N_DEV = 8
NR = 5
NE = 4


def kernel(x, w_mat):
    m_per, k = x.shape
    _, n_per = w_mat.shape
    p = m_per // 2

    def body(x_ref, w_ref, out_ref, xg_ref, w_vmem, fs, fr, bs, br, es, er,
             cp_sems):
        l = lax.axis_index("i")
        pos = jnp.where(l < 4, l, 11 - l)
        parity = lax.rem(pos, 2)
        s = 1 - 2 * parity

        def l_of(qq):
            qq = lax.rem(qq + 2 * N_DEV, N_DEV)
            return jnp.where(qq < 4, qq, 11 - qq)

        right = l_of(pos + 1)
        left = l_of(pos - 1)
        prt = l_of(pos + 3 * s)

        def o(kk):
            return l_of(pos - kk)

        def q_(kk):
            return l_of(pos + kk)

        def piece(pidx):
            return xg_ref.at[pl.ds(pidx * p, p), :]

        def send(src, pidx, dev, ssem, rsem):
            d = pltpu.make_async_remote_copy(
                src_ref=src, dst_ref=piece(pidx),
                send_sem=ssem, recv_sem=rsem,
                device_id=(dev,), device_id_type=pl.DeviceIdType.MESH,
            )
            d.start()
            return d

        def recv_wait(pidx, ssem, rsem):
            pltpu.make_async_remote_copy(
                src_ref=piece(pidx), dst_ref=piece(pidx),
                send_sem=ssem, recv_sem=rsem,
                device_id=(right,), device_id_type=pl.DeviceIdType.MESH,
            ).wait_recv()

        def gemm_piece(pidx):
            out_ref[pl.ds(pidx * p, p), :] = jnp.maximum(
                jnp.dot(piece(pidx)[...], w_vmem[...],
                        preferred_element_type=jnp.float32),
                0.0,
            )

        x_cp = pltpu.make_async_copy(
            x_ref, xg_ref.at[pl.ds(l * m_per, m_per), :], cp_sems.at[0])
        x_cp.start()
        w_cp = pltpu.make_async_copy(w_ref, w_vmem, cp_sems.at[1])
        w_cp.start()

        barrier_sem = pltpu.get_barrier_semaphore()
        for nbr in (left, right, prt):
            pl.semaphore_signal(
                barrier_sem, inc=1,
                device_id=(nbr,), device_id_type=pl.DeviceIdType.MESH,
            )
        pl.semaphore_wait(barrier_sem, 3)

        fsendp = [2 * o(0), 2 * o(0) + 1, 2 * o(1), 2 * o(1) + 1, 2 * o(2)]
        frecvp = [2 * o(1), 2 * o(1) + 1, 2 * o(2), 2 * o(2) + 1, 2 * o(3)]
        bsendp = [2 * q_(0) + 1, 2 * q_(0), 2 * q_(1) + 1, 2 * q_(1),
                  2 * q_(2) + 1]
        brecvp = [2 * q_(1) + 1, 2 * q_(1), 2 * q_(2) + 1, 2 * q_(2),
                  2 * q_(3) + 1]
        c1 = l_of(pos - s)
        c2 = l_of(pos - 2 * s)
        esendp = [2 * l + 1 - parity,
                  2 * c1 + parity,
                  2 * c1 + 1 - parity,
                  2 * c2 + parity]
        a = l_of(pos + 4)
        erecvp = [2 * prt + parity,
                  2 * a + 1 - parity,
                  2 * a + parity,
                  2 * l_of(pos - 3 * s) + 1 - parity]

        started = []
        x_cp.wait()
        started.append(send(piece(fsendp[0]), fsendp[0], right,
                            fs.at[0], fr.at[0]))
        started.append(send(piece(fsendp[1]), fsendp[1], right,
                            fs.at[1], fr.at[1]))
        started.append(send(piece(bsendp[0]), bsendp[0], left,
                            bs.at[0], br.at[0]))
        started.append(send(piece(bsendp[1]), bsendp[1], left,
                            bs.at[1], br.at[1]))
        started.append(send(piece(esendp[0]), esendp[0], prt,
                            es.at[0], er.at[0]))

        w_cp.wait()
        out_ref[pl.ds(l * m_per, m_per), :] = jnp.maximum(
            jnp.dot(xg_ref[pl.ds(l * m_per, m_per), :], w_vmem[...],
                    preferred_element_type=jnp.float32),
            0.0,
        )

        for j in range(NR):
            recv_wait(frecvp[j], fs.at[j], fr.at[j])
            if j + 2 < NR:
                started.append(send(piece(fsendp[j + 2]), fsendp[j + 2],
                                    right, fs.at[j + 2], fr.at[j + 2]))
            recv_wait(brecvp[j], bs.at[j], br.at[j])
            if j + 2 < NR:
                started.append(send(piece(bsendp[j + 2]), bsendp[j + 2],
                                    left, bs.at[j + 2], br.at[j + 2]))
            if j < 3:
                started.append(send(piece(esendp[j + 1]), esendp[j + 1],
                                    prt, es.at[j + 1], er.at[j + 1]))
            gemm_piece(frecvp[j])
            gemm_piece(brecvp[j])

        for t in range(NE):
            recv_wait(erecvp[t], es.at[t], er.at[t])
            gemm_piece(erecvp[t])

        for d in started:
            d.wait_send()

    return pl.pallas_call(
        body,
        out_shape=jax.ShapeDtypeStruct((N_DEV * m_per, n_per), jnp.float32),
        in_specs=[
            pl.BlockSpec(memory_space=pltpu.MemorySpace.HBM),
            pl.BlockSpec(memory_space=pltpu.MemorySpace.HBM),
        ],
        out_specs=pl.BlockSpec(memory_space=pltpu.VMEM),
        scratch_shapes=[
            pltpu.VMEM((N_DEV * m_per, k), jnp.float32),
            pltpu.VMEM((k, n_per), jnp.float32),
            pltpu.SemaphoreType.DMA((NR,)),
            pltpu.SemaphoreType.DMA((NR,)),
            pltpu.SemaphoreType.DMA((NR,)),
            pltpu.SemaphoreType.DMA((NR,)),
            pltpu.SemaphoreType.DMA((NE,)),
            pltpu.SemaphoreType.DMA((NE,)),
            pltpu.SemaphoreType.DMA((2,)),
        ],
        compiler_params=pltpu.CompilerParams(collective_id=0),
    )(x, w_mat)
